# speedup vs baseline: 1.2232x; 1.2232x over previous
import jax
import jax.numpy as jnp
from jax.experimental import pallas as pl


def _combine_body(proj_z_ref, h_ref, esum_ref, gfc_ref, out_ref):
    esum = jnp.maximum(esum_ref[...], 1e-16)
    h = h_ref[...] / esum
    g = jax.nn.sigmoid(gfc_ref[...])
    out_ref[...] = proj_z_ref[...] + g * h


def kernel(v, proj_z, edge_index, Wa, att_l, att_r, gate_l, gate_m, gate_r, Wgm):
    src = edge_index[0]
    dst = edge_index[1]
    n = v.shape[0]
    al2 = (att_l @ Wa)[0]
    ar2 = (att_r @ Wa)[0]
    el = v @ al2
    er = v @ ar2
    e = jax.nn.leaky_relu(el[src] + er[dst], negative_slope=0.01)
    ex = jnp.exp(e)
    esum = jax.ops.segment_sum(ex, dst, num_segments=n)
    h_acc = jax.ops.segment_sum(ex[:, None] * proj_z[src], dst, num_segments=n)
    gp = v @ Wgm.T
    max_feat = jax.ops.segment_max(gp[src], dst, num_segments=n)
    max_feat = jnp.where(jnp.isneginf(max_feat), 0.0, max_feat)
    deg = jax.ops.segment_sum(jnp.ones((src.shape[0],), jnp.float32), dst,
                              num_segments=n)
    vr = v @ gate_r[0]
    vl = v @ gate_l[0]
    mean_r = jax.ops.segment_sum(vr[src], dst, num_segments=n) / jnp.maximum(deg, 1.0)
    gfc = vl + max_feat @ gate_m[0] + mean_r
    out = pl.pallas_call(
        _combine_body,
        out_shape=jax.ShapeDtypeStruct((n, v.shape[1]), jnp.float32),
    )(proj_z, h_acc, esum[:, None], gfc[:, None])
    return out


# speedup vs baseline: 5.9415x; 4.8573x over previous
import functools

import jax
import jax.numpy as jnp
from jax import lax
from jax.experimental import pallas as pl
from jax.experimental.pallas import tpu as pltpu
from jax.experimental.pallas import tpu_sc as plsc

N = 10000
E = 320000
D = 128
NT = 32
BKT = 313
EB = 2000
NB = E // EB
CAP = EB + 16


def _pre_body(v_ref, pz_ref, wgm_ref, m8_ref, tbl_ref, scal_ref):
    vb = v_ref[...]
    gp = lax.dot_general(vb, wgm_ref[...], (((1,), (1,)), ((), ())),
                         preferred_element_type=jnp.float32)
    tbl_ref[:, 0:D] = pz_ref[...]
    tbl_ref[:, D:2 * D] = gp
    scal_ref[...] = lax.dot_general(vb, m8_ref[...], (((1,), (1,)), ((), ())),
                                    preferred_element_type=jnp.float32)


def _post_body(pz_ref, h_ref, mx_ref, es_ref, dg_ref, vs_ref, vl_ref, gm_ref,
               out_ref):
    es = jnp.maximum(es_ref[...], 1e-16)
    dg = dg_ref[...]
    h = h_ref[...] / es
    mx = jnp.where(dg > 0.0, mx_ref[...], 0.0)
    mdot = jnp.sum(mx * gm_ref[...], axis=1, keepdims=True)
    mean_r = vs_ref[...] / jnp.maximum(dg, 1.0)
    gfc = vl_ref[...] + mdot + mean_r
    out_ref[...] = pz_ref[...] + jax.nn.sigmoid(gfc) * h


def _sc_body(src_hbm, dst_hbm, el_hbm, er_hbm, vr_hbm, tbl_hbm,
             h_out, mx_out, es_out, dg_out, vs_out,
             h_acc, mx_acc, esum, deg, vrs, el_t, er_t, vr_t,
             sblk, dblk, srcc, dstc, exc, rows):
    wid = lax.axis_index("s") * 2 + lax.axis_index("c")
    pltpu.sync_copy(el_hbm, el_t)
    pltpu.sync_copy(er_hbm, er_t)
    pltpu.sync_copy(vr_hbm, vr_t)

    zf = jnp.zeros((16,), jnp.float32)
    zi = jnp.zeros((16,), jnp.int32)
    ninf = jnp.full((16,), -3.4e38, jnp.float32)
    ones = jnp.ones((16,), jnp.float32)

    @pl.loop(0, BKT)
    def _(r):
        @pl.loop(0, D, step=16, unroll=True)
        def _(c):
            h_acc[r, pl.ds(c, 16)] = zf
            mx_acc[r, pl.ds(c, 16)] = ninf

    @pl.loop(0, 320, step=16)
    def _(i):
        esum[pl.ds(i, 16)] = zf
        deg[pl.ds(i, 16)] = zf
        vrs[pl.ds(i, 16)] = zf

    @pl.loop(0, CAP, step=16)
    def _(i):
        srcc[pl.ds(i, 16)] = zi
        dstc[pl.ds(i, 16)] = zi

    lanes = lax.iota(jnp.int32, 16)

    @pl.loop(0, NB)
    def _(blk):
        off = blk * EB
        pltpu.sync_copy(src_hbm.at[pl.ds(off, EB)], sblk)
        pltpu.sync_copy(dst_hbm.at[pl.ds(off, EB)], dblk)

        @pl.loop(0, EB, step=16, init_carry=jnp.int32(0))
        def filt(i, cnt):
            dvec = dblk[pl.ds(i, 16)]
            m = (dvec & 31) == wid
            svec = sblk[pl.ds(i, 16)]
            plsc.store_compressed(srcc.at[pl.ds(cnt, 16)], svec, mask=m)
            plsc.store_compressed(dstc.at[pl.ds(cnt, 16)], dvec, mask=m)
            c = plsc.all_reduce_population_count(m)
            return cnt + c[0]

        cnt = filt
        ngrp = (cnt + 15) >> 4

        @pl.loop(0, ngrp)
        def _(g):
            base = g * 16
            svec = srcc[pl.ds(base, 16)]
            dvec = dstc[pl.ds(base, 16)]
            els = plsc.load_gather(el_t, [svec])
            erd = plsc.load_gather(er_t, [dvec])
            e = els + erd
            e = jnp.maximum(e, e * 0.01)
            ex = jnp.exp(e)
            valid = (base + lanes) < cnt
            lvec = jax.lax.shift_right_logical(dvec, 5)
            plsc.addupdate_scatter(esum, [lvec], ex, mask=valid)
            plsc.addupdate_scatter(deg, [lvec], ones, mask=valid)
            vrv = plsc.load_gather(vr_t, [svec])
            plsc.addupdate_scatter(vrs, [lvec], vrv, mask=valid)
            exc[pl.ds(base, 16)] = ex
            pltpu.sync_copy(tbl_hbm.at[svec], rows)
            rem = jnp.minimum(cnt - base, 16)

            @pl.loop(0, rem)
            def _(e2):
                d = dstc[pl.ds(base + e2, 16)][0]
                li = jax.lax.shift_right_logical(d, 5)
                exe = exc[pl.ds(base + e2, 16)][0]
                for j in range(D // 16):
                    c0 = j * 16
                    h_acc[li, pl.ds(c0, 16)] = (
                        h_acc[li, pl.ds(c0, 16)] + exe * rows[e2, pl.ds(c0, 16)])
                    mx_acc[li, pl.ds(c0, 16)] = jnp.maximum(
                        mx_acc[li, pl.ds(c0, 16)], rows[e2, pl.ds(D + c0, 16)])

    pltpu.sync_copy(h_acc, h_out.at[wid])
    pltpu.sync_copy(mx_acc, mx_out.at[wid])
    pltpu.sync_copy(esum, es_out.at[wid])
    pltpu.sync_copy(deg, dg_out.at[wid])
    pltpu.sync_copy(vrs, vs_out.at[wid])


@jax.jit
def kernel(v, proj_z, edge_index, Wa, att_l, att_r, gate_l, gate_m, gate_r, Wgm):
    al2 = att_l @ Wa
    ar2 = att_r @ Wa
    m8 = jnp.concatenate(
        [al2, ar2, gate_r, gate_l, jnp.zeros((4, D), jnp.float32)], axis=0)

    nblk = 10
    rows_per = N // nblk
    tbl, scal = pl.pallas_call(
        _pre_body,
        grid=(nblk,),
        in_specs=[
            pl.BlockSpec((rows_per, D), lambda i: (i, 0)),
            pl.BlockSpec((rows_per, D), lambda i: (i, 0)),
            pl.BlockSpec((D, D), lambda i: (0, 0)),
            pl.BlockSpec((8, D), lambda i: (0, 0)),
        ],
        out_specs=[
            pl.BlockSpec((rows_per, 2 * D), lambda i: (i, 0)),
            pl.BlockSpec((rows_per, 8), lambda i: (i, 0)),
        ],
        out_shape=[
            jax.ShapeDtypeStruct((N, 2 * D), jnp.float32),
            jax.ShapeDtypeStruct((N, 8), jnp.float32),
        ],
    )(v, proj_z, Wgm, m8)

    el = scal[:, 0]
    er = scal[:, 1]
    vr = scal[:, 2]
    vl = scal[:, 3:4]
    src = edge_index[0]
    dst = edge_index[1]

    mesh = plsc.VectorSubcoreMesh(core_axis_name="c", subcore_axis_name="s")
    sc = pl.kernel(
        _sc_body,
        compiler_params=pltpu.CompilerParams(needs_layout_passes=False),
        out_type=[
            jax.ShapeDtypeStruct((NT, BKT, D), jnp.float32),
            jax.ShapeDtypeStruct((NT, BKT, D), jnp.float32),
            jax.ShapeDtypeStruct((NT, 320), jnp.float32),
            jax.ShapeDtypeStruct((NT, 320), jnp.float32),
            jax.ShapeDtypeStruct((NT, 320), jnp.float32),
        ],
        mesh=mesh,
        scratch_types=[
            pltpu.VMEM((BKT, D), jnp.float32),
            pltpu.VMEM((BKT, D), jnp.float32),
            pltpu.VMEM((320,), jnp.float32),
            pltpu.VMEM((320,), jnp.float32),
            pltpu.VMEM((320,), jnp.float32),
            pltpu.VMEM((N,), jnp.float32),
            pltpu.VMEM((N,), jnp.float32),
            pltpu.VMEM((N,), jnp.float32),
            pltpu.VMEM((EB,), jnp.int32),
            pltpu.VMEM((EB,), jnp.int32),
            pltpu.VMEM((CAP,), jnp.int32),
            pltpu.VMEM((CAP,), jnp.int32),
            pltpu.VMEM((CAP,), jnp.float32),
            pltpu.VMEM((16, 2 * D), jnp.float32),
        ],
    )
    h_out, mx_out, es_out, dg_out, vs_out = sc(src, dst, el, er, vr, tbl)

    h_full = h_out.transpose(1, 0, 2).reshape(NT * BKT, D)[:N]
    mx_full = mx_out.transpose(1, 0, 2).reshape(NT * BKT, D)[:N]
    es_full = es_out[:, :BKT].T.reshape(NT * BKT)[:N, None]
    dg_full = dg_out[:, :BKT].T.reshape(NT * BKT)[:N, None]
    vs_full = vs_out[:, :BKT].T.reshape(NT * BKT)[:N, None]

    out = pl.pallas_call(
        _post_body,
        grid=(nblk,),
        in_specs=[
            pl.BlockSpec((rows_per, D), lambda i: (i, 0)),
            pl.BlockSpec((rows_per, D), lambda i: (i, 0)),
            pl.BlockSpec((rows_per, D), lambda i: (i, 0)),
            pl.BlockSpec((rows_per, 1), lambda i: (i, 0)),
            pl.BlockSpec((rows_per, 1), lambda i: (i, 0)),
            pl.BlockSpec((rows_per, 1), lambda i: (i, 0)),
            pl.BlockSpec((rows_per, 1), lambda i: (i, 0)),
            pl.BlockSpec((1, D), lambda i: (0, 0)),
        ],
        out_specs=pl.BlockSpec((rows_per, D), lambda i: (i, 0)),
        out_shape=jax.ShapeDtypeStruct((N, D), jnp.float32),
    )(proj_z, h_full, mx_full, es_full, dg_full, vs_full, vl, gate_m)
    return out
